# baseline (speedup 1.0000x reference)
import functools

import jax
import jax.numpy as jnp
from jax import lax
from jax.experimental import pallas as pl
from jax.experimental.pallas import tpu as pltpu
from jax.experimental.pallas import tpu_sc as plsc

_NUM_CORES = 2
_NUM_SUBCORES = 16
_NW = _NUM_CORES * _NUM_SUBCORES
_CHUNK = 32
_NBUF = 4


@functools.lru_cache(maxsize=None)
def _build_embed(rows, cols, vocab, d_model):
    mesh = plsc.VectorSubcoreMesh(core_axis_name="c", subcore_axis_name="s")
    b_per_w = rows * cols // _NW
    n_chunks = b_per_w // _CHUNK
    w_per_row = cols // b_per_w

    @functools.partial(
        pl.kernel,
        mesh=mesh,
        out_type=jax.ShapeDtypeStruct((rows, cols, d_model), jnp.float32),
        scratch_types=(
            [pltpu.VMEM((b_per_w,), jnp.int32)]
            + [pltpu.VMEM((_CHUNK, d_model), jnp.float32) for _ in range(_NBUF)]
            + [pltpu.SemaphoreType.DMA for _ in range(2 * _NBUF)]
        ),
    )
    def embed(idx_hbm, table_hbm, out_hbm, idx_v, *rest):
        bufs = rest[:_NBUF]
        gsems = rest[_NBUF:2 * _NBUF]
        wsems = rest[2 * _NBUF:]
        wid = lax.axis_index("s") * _NUM_CORES + lax.axis_index("c")
        row = wid // w_per_row
        col0 = (wid % w_per_row) * b_per_w

        pltpu.sync_copy(idx_hbm.at[row, pl.ds(col0, b_per_w)], idx_v)

        def start_gather(j):
            return pltpu.async_copy(
                table_hbm.at[idx_v.at[pl.ds(j * _CHUNK, _CHUNK)]],
                bufs[j % _NBUF], gsems[j % _NBUF])

        def start_write(j):
            return pltpu.async_copy(
                bufs[j % _NBUF],
                out_hbm.at[row, pl.ds(col0 + j * _CHUNK, _CHUNK)],
                wsems[j % _NBUF])

        gathers = [None] * n_chunks
        writes = [None] * n_chunks
        for j in range(min(_NBUF - 1, n_chunks)):
            gathers[j] = start_gather(j)
        for j in range(n_chunks):
            nxt = j + _NBUF - 1
            if nxt < n_chunks:
                if nxt - _NBUF >= 0:
                    writes[nxt - _NBUF].wait()
                gathers[nxt] = start_gather(nxt)
            gathers[j].wait()
            writes[j] = start_write(j)
        for j in range(max(0, n_chunks - _NBUF), n_chunks):
            writes[j].wait()

    return embed


def kernel(tokens, W_E):
    rows, cols = tokens.shape
    assert (rows * cols) % (_NW * _CHUNK) == 0 and cols % (rows * cols // _NW) == 0
    return _build_embed(rows, cols, W_E.shape[0], W_E.shape[1])(tokens, W_E)


# speedup vs baseline: 1.0136x; 1.0136x over previous
import functools

import jax
import jax.numpy as jnp
from jax import lax
from jax.experimental import pallas as pl
from jax.experimental.pallas import tpu as pltpu
from jax.experimental.pallas import tpu_sc as plsc

_NUM_CORES = 2
_NUM_SUBCORES = 16
_NW = _NUM_CORES * _NUM_SUBCORES
_CHUNK = 32
_NBUF = 5


@functools.lru_cache(maxsize=None)
def _build_embed(rows, cols, vocab, d_model):
    mesh = plsc.VectorSubcoreMesh(core_axis_name="c", subcore_axis_name="s")
    b_per_w = rows * cols // _NW
    n_chunks = b_per_w // _CHUNK
    w_per_row = cols // b_per_w

    @functools.partial(
        pl.kernel,
        mesh=mesh,
        out_type=jax.ShapeDtypeStruct((rows, cols, d_model), jnp.float32),
        scratch_types=(
            [pltpu.VMEM((b_per_w,), jnp.int32)]
            + [pltpu.VMEM((_CHUNK, d_model), jnp.float32) for _ in range(_NBUF)]
            + [pltpu.SemaphoreType.DMA for _ in range(2 * _NBUF)]
        ),
    )
    def embed(idx_hbm, table_hbm, out_hbm, idx_v, *rest):
        bufs = rest[:_NBUF]
        gsems = rest[_NBUF:2 * _NBUF]
        wsems = rest[2 * _NBUF:]
        wid = lax.axis_index("s") * _NUM_CORES + lax.axis_index("c")
        row = wid // w_per_row
        col0 = (wid % w_per_row) * b_per_w

        pltpu.sync_copy(idx_hbm.at[row, pl.ds(col0, b_per_w)], idx_v)

        def start_gather(j):
            return pltpu.async_copy(
                table_hbm.at[idx_v.at[pl.ds(j * _CHUNK, _CHUNK)]],
                bufs[j % _NBUF], gsems[j % _NBUF])

        def start_write(j):
            return pltpu.async_copy(
                bufs[j % _NBUF],
                out_hbm.at[row, pl.ds(col0 + j * _CHUNK, _CHUNK)],
                wsems[j % _NBUF])

        gathers = [None] * n_chunks
        writes = [None] * n_chunks
        for j in range(min(_NBUF - 1, n_chunks)):
            gathers[j] = start_gather(j)
        for j in range(n_chunks):
            nxt = j + _NBUF - 1
            if nxt < n_chunks:
                if nxt - _NBUF >= 0:
                    writes[nxt - _NBUF].wait()
                gathers[nxt] = start_gather(nxt)
            gathers[j].wait()
            writes[j] = start_write(j)
        for j in range(max(0, n_chunks - _NBUF), n_chunks):
            writes[j].wait()

    return embed


def kernel(tokens, W_E):
    rows, cols = tokens.shape
    assert (rows * cols) % (_NW * _CHUNK) == 0 and cols % (rows * cols // _NW) == 0
    return _build_embed(rows, cols, W_E.shape[0], W_E.shape[1])(tokens, W_E)
